# baseline (speedup 1.0000x reference)
import functools

import jax
import jax.numpy as jnp
from jax import lax
from jax.experimental import pallas as pl
from jax.experimental.pallas import tpu as pltpu
from jax.experimental.pallas import tpu_sc as plsc

B = 16384
D = 64
N_ROWS = 1000000
NPAIR = N_ROWS // 2
NC = 2
NS = 16
NW = NC * NS
BPW = B // NW
NPASS = 2
PB = BPW // NPASS
CHUNK = 128
NCHUNK = PB // CHUNK
BLK = 16


def _nrsqrt(y):
    i = lax.bitcast_convert_type(y, jnp.int32)
    i = jnp.int32(0x5F3759DF) - lax.shift_right_arithmetic(i, 1)
    x = lax.bitcast_convert_type(i, jnp.float32)
    for _ in range(3):
        x = x * (1.5 - 0.5 * y * x * x)
    return x


def _gmf_body(job_hbm, geek_hbm, jemb_hbm, gemb_hbm, w_hbm, out_hbm,
              idx_j, idx_g, tidx_j, tidx_g, rows_j, rows_g, w_v, out_v, sem):
    wid = lax.axis_index("s") * NC + lax.axis_index("c")
    base = wid * BPW

    pltpu.sync_copy(w_hbm, w_v)
    pltpu.sync_copy(job_hbm.at[pl.ds(base, BPW)], idx_j)
    pltpu.sync_copy(geek_hbm.at[pl.ds(base, BPW)], idx_g)

    wv = [w_v[pl.ds(c * 16, 16)] for c in range(D // 16)]
    lane = lax.iota(jnp.int32, 16)
    perms = [jnp.bitwise_xor(lane, s) for s in (8, 4, 2, 1)]

    gdn = lax.GatherDimensionNumbers(
        offset_dims=(), collapsed_slice_dims=(0,), start_index_map=(0,))

    def lanesum(v):
        for p in perms:
            v = v + lax.gather(v, p[:, None], gdn, (1,),
                               mode=lax.GatherScatterMode.PROMISE_IN_BOUNDS)
        return v

    def one_pass(p, pcarry):
        pbase = p * PB

        for c in range(PB // 16):
            jv = idx_j[pl.ds(pbase + c * 16, 16)]
            tidx_j[c // 8, pl.ds((c % 8) * 16, 16)] = (
                lax.shift_right_logical(jv, 1))
            gv = idx_g[pl.ds(pbase + c * 16, 16)]
            tidx_g[c // 8, pl.ds((c % 8) * 16, 16)] = (
                lax.shift_right_logical(gv, 1))

        cps = []
        for c in range(NCHUNK):
            cps.append(pltpu.async_copy(
                jemb_hbm.at[tidx_j.at[c]],
                rows_j.at[pl.ds(c * CHUNK, CHUNK)], sem))
            cps.append(pltpu.async_copy(
                gemb_hbm.at[tidx_g.at[c]],
                rows_g.at[pl.ds(c * CHUNK, CHUNK)], sem))
        for cp in cps:
            cp.wait()

        def block(blk, carry):
            row0 = blk * BLK
            hj = jnp.bitwise_and(idx_j[pl.ds(pbase + row0, 16)], 1) * D
            hg = jnp.bitwise_and(idx_g[pl.ds(pbase + row0, 16)], 1) * D
            vx = jnp.zeros((16,), jnp.float32)
            vjj = jnp.zeros((16,), jnp.float32)
            vgg = jnp.zeros((16,), jnp.float32)
            for r in range(BLK):
                row = row0 + r
                px = jnp.zeros((16,), jnp.float32)
                pjj = jnp.zeros((16,), jnp.float32)
                pgg = jnp.zeros((16,), jnp.float32)
                for c in range(D // 16):
                    jvv = rows_j[row, pl.ds(hj[r] + c * 16, 16)]
                    gvv = rows_g[row, pl.ds(hg[r] + c * 16, 16)]
                    px = px + jvv * gvv * wv[c]
                    pjj = pjj + jvv * jvv
                    pgg = pgg + gvv * gvv
                m = lane == r
                vx = jnp.where(m, lanesum(px), vx)
                vjj = jnp.where(m, lanesum(pjj), vjj)
                vgg = jnp.where(m, lanesum(pgg), vgg)
            out_v[pl.ds(pbase + row0, BLK)] = vx * _nrsqrt(vjj * vgg)
            return carry

        lax.fori_loop(0, PB // BLK, block, 0)
        return pcarry

    lax.fori_loop(0, NPASS, one_pass, 0)
    pltpu.sync_copy(out_v, out_hbm.at[pl.ds(base, BPW)])


@functools.partial(jax.jit, static_argnums=())
def _gmf(job_f, geek_f, jpair, gpair, w_f):
    mesh = plsc.VectorSubcoreMesh(core_axis_name="c", subcore_axis_name="s")
    run = functools.partial(
        pl.kernel,
        out_type=jax.ShapeDtypeStruct((B,), jnp.float32),
        mesh=mesh,
        scratch_types=[
            pltpu.VMEM((BPW,), jnp.int32),
            pltpu.VMEM((BPW,), jnp.int32),
            pltpu.VMEM((NCHUNK, CHUNK), jnp.int32),
            pltpu.VMEM((NCHUNK, CHUNK), jnp.int32),
            pltpu.VMEM((PB, 2 * D), jnp.float32),
            pltpu.VMEM((PB, 2 * D), jnp.float32),
            pltpu.VMEM((D,), jnp.float32),
            pltpu.VMEM((BPW,), jnp.float32),
            pltpu.SemaphoreType.DMA,
        ],
    )(_gmf_body)
    return run(job_f, geek_f, jpair, gpair, w_f)


def kernel(job, geek, job_emb, geek_emb, W):
    job_f = job.reshape(-1).astype(jnp.int32)
    geek_f = geek.reshape(-1).astype(jnp.int32)
    w_f = W.reshape(-1)
    jpair = job_emb.reshape(NPAIR, 2 * D)
    gpair = geek_emb.reshape(NPAIR, 2 * D)
    out = _gmf(job_f, geek_f, jpair, gpair, w_f)
    return out.reshape(B, 1)


# speedup vs baseline: 1.3561x; 1.3561x over previous
import functools

import jax
import jax.numpy as jnp
from jax import lax
from jax.experimental import pallas as pl
from jax.experimental.pallas import tpu as pltpu
from jax.experimental.pallas import tpu_sc as plsc

B = 16384
D = 64
SPLIT = 8192
NC = 2
NS = 16
NW = NC * NS
BPW = SPLIT // NW
BLK = 16
TCB = 128


def _nrsqrt(y):
    i = lax.bitcast_convert_type(y, jnp.int32)
    i = jnp.int32(0x5F3759DF) - lax.shift_right_arithmetic(i, 1)
    x = lax.bitcast_convert_type(i, jnp.float32)
    for _ in range(3):
        x = x * (1.5 - 0.5 * y * x * x)
    return x


def _sc_body(job_hbm, geek_hbm, jemb_hbm, gemb_hbm, w_hbm, out_hbm,
             idx_j, idx_g, rows_j, rows_g, w_v, out_v, sem):
    wid = lax.axis_index("s") * NC + lax.axis_index("c")
    base = wid * BPW

    pltpu.sync_copy(w_hbm, w_v)
    pltpu.sync_copy(job_hbm.at[pl.ds(base, BPW)], idx_j)
    pltpu.sync_copy(geek_hbm.at[pl.ds(base, BPW)], idx_g)

    wv = [w_v[pl.ds(c * 16, 16)] for c in range(D // 16)]
    lane = lax.iota(jnp.int32, 16)
    perms = [jnp.bitwise_xor(lane, s) for s in (8, 4, 2, 1)]

    gdn = lax.GatherDimensionNumbers(
        offset_dims=(), collapsed_slice_dims=(0,), start_index_map=(0,))

    def lanesum(v):
        for p in perms:
            v = v + lax.gather(v, p[:, None], gdn, (1,),
                               mode=lax.GatherScatterMode.PROMISE_IN_BOUNDS)
        return v

    @plsc.parallel_loop(0, BPW // 16)
    def fire_j(g):
        row0 = g * 16
        jv = idx_j[pl.ds(row0, 16)]
        for ln in range(16):
            pltpu.make_async_copy(
                jemb_hbm.at[jv[ln]], rows_j.at[row0 + ln], sem).start()

    @plsc.parallel_loop(0, BPW // 16)
    def fire_g(g):
        row0 = g * 16
        gv = idx_g[pl.ds(row0, 16)]
        for ln in range(16):
            pltpu.make_async_copy(
                gemb_hbm.at[gv[ln]], rows_g.at[row0 + ln], sem).start()

    pltpu.make_async_copy(jemb_hbm.at[pl.ds(0, BPW)], rows_j, sem).wait()
    pltpu.make_async_copy(gemb_hbm.at[pl.ds(0, BPW)], rows_g, sem).wait()

    def block(blk, carry):
        row0 = blk * BLK
        vx = jnp.zeros((16,), jnp.float32)
        vjj = jnp.zeros((16,), jnp.float32)
        vgg = jnp.zeros((16,), jnp.float32)
        for r in range(BLK):
            row = row0 + r
            px = jnp.zeros((16,), jnp.float32)
            pjj = jnp.zeros((16,), jnp.float32)
            pgg = jnp.zeros((16,), jnp.float32)
            for c in range(D // 16):
                jvv = rows_j[row, pl.ds(c * 16, 16)]
                gvv = rows_g[row, pl.ds(c * 16, 16)]
                px = px + jvv * gvv * wv[c]
                pjj = pjj + jvv * jvv
                pgg = pgg + gvv * gvv
            m = lane == r
            vx = jnp.where(m, lanesum(px), vx)
            vjj = jnp.where(m, lanesum(pjj), vjj)
            vgg = jnp.where(m, lanesum(pgg), vgg)
        out_v[pl.ds(row0, BLK)] = vx * _nrsqrt(vjj * vgg)
        return carry

    lax.fori_loop(0, BPW // BLK, block, 0)
    pltpu.sync_copy(out_v, out_hbm.at[pl.ds(base, BPW)])


def _tc_body(jidx_sm, gidx_sm, jemb_any, gemb_any, w_v, out_b,
             rows_j, rows_g, sem):
    i = pl.program_id(0)
    base = i * TCB
    cps = []
    for r in range(TCB):
        cps.append(pltpu.make_async_copy(
            jemb_any.at[jidx_sm[base + r]], rows_j.at[r], sem))
        cps.append(pltpu.make_async_copy(
            gemb_any.at[gidx_sm[base + r]], rows_g.at[r], sem))
    for cp in cps:
        cp.start()
    for cp in cps:
        cp.wait()
    j = rows_j[...]
    g = rows_g[...]
    x = jnp.sum(j * g * w_v[...], axis=1)
    n2 = jnp.sum(j * j, axis=1) * jnp.sum(g * g, axis=1)
    out_b[...] = (x * lax.rsqrt(n2))[:, None]


@functools.partial(jax.jit, static_argnums=())
def _gmf(job_f, geek_f, job_emb, geek_emb, w_f):
    mesh = plsc.VectorSubcoreMesh(core_axis_name="c", subcore_axis_name="s")
    sc_run = functools.partial(
        pl.kernel,
        out_type=jax.ShapeDtypeStruct((SPLIT,), jnp.float32),
        mesh=mesh,
        scratch_types=[
            pltpu.VMEM((BPW,), jnp.int32),
            pltpu.VMEM((BPW,), jnp.int32),
            pltpu.VMEM((BPW, D), jnp.float32),
            pltpu.VMEM((BPW, D), jnp.float32),
            pltpu.VMEM((D,), jnp.float32),
            pltpu.VMEM((BPW,), jnp.float32),
            pltpu.SemaphoreType.DMA,
        ],
    )(_sc_body)
    out_sc = sc_run(job_f[:SPLIT], geek_f[:SPLIT], job_emb, geek_emb, w_f)

    ntc = B - SPLIT
    out_tc = pl.pallas_call(
        _tc_body,
        grid=(ntc // TCB,),
        in_specs=[
            pl.BlockSpec(memory_space=pltpu.SMEM),
            pl.BlockSpec(memory_space=pltpu.SMEM),
            pl.BlockSpec(memory_space=pltpu.HBM),
            pl.BlockSpec(memory_space=pltpu.HBM),
            pl.BlockSpec((1, D), lambda i: (0, 0)),
        ],
        out_specs=pl.BlockSpec((TCB, 1), lambda i: (i, 0)),
        out_shape=jax.ShapeDtypeStruct((ntc, 1), jnp.float32),
        scratch_shapes=[
            pltpu.VMEM((TCB, D), jnp.float32),
            pltpu.VMEM((TCB, D), jnp.float32),
            pltpu.SemaphoreType.DMA,
        ],
    )(job_f[SPLIT:], geek_f[SPLIT:], job_emb, geek_emb,
      w_f.reshape(1, D))
    return out_sc, out_tc


def kernel(job, geek, job_emb, geek_emb, W):
    job_f = job.reshape(-1).astype(jnp.int32)
    geek_f = geek.reshape(-1).astype(jnp.int32)
    w_f = W.reshape(-1)
    out_sc, out_tc = _gmf(job_f, geek_f, job_emb, geek_emb, w_f)
    return jnp.concatenate([out_sc.reshape(SPLIT, 1), out_tc], axis=0)


# speedup vs baseline: 1.3574x; 1.0010x over previous
import functools

import jax
import jax.numpy as jnp
from jax import lax
from jax.experimental import pallas as pl
from jax.experimental.pallas import tpu as pltpu
from jax.experimental.pallas import tpu_sc as plsc

B = 16384
D = 64
SPLIT = 8192
NC = 2
NS = 16
NW = NC * NS
BPW = SPLIT // NW
BLK = 16
TCB = 128


def _nrsqrt(y):
    i = lax.bitcast_convert_type(y, jnp.int32)
    i = jnp.int32(0x5F3759DF) - lax.shift_right_arithmetic(i, 1)
    x = lax.bitcast_convert_type(i, jnp.float32)
    for _ in range(3):
        x = x * (1.5 - 0.5 * y * x * x)
    return x


def _sc_body(job_hbm, geek_hbm, jemb_hbm, gemb_hbm, w_hbm, out_hbm,
             idx_j, idx_g, rows_j, rows_g, w_v, out_v, sem):
    wid = lax.axis_index("s") * NC + lax.axis_index("c")
    base = wid * BPW

    pltpu.sync_copy(w_hbm, w_v)
    pltpu.sync_copy(job_hbm.at[pl.ds(base, BPW)], idx_j)
    pltpu.sync_copy(geek_hbm.at[pl.ds(base, BPW)], idx_g)

    wv = [w_v[pl.ds(c * 16, 16)] for c in range(D // 16)]
    lane = lax.iota(jnp.int32, 16)
    perms = [jnp.bitwise_xor(lane, s) for s in (8, 4, 2, 1)]

    gdn = lax.GatherDimensionNumbers(
        offset_dims=(), collapsed_slice_dims=(0,), start_index_map=(0,))

    def lanesum(v):
        for p in perms:
            v = v + lax.gather(v, p[:, None], gdn, (1,),
                               mode=lax.GatherScatterMode.PROMISE_IN_BOUNDS)
        return v

    @plsc.parallel_loop(0, BPW // 16)
    def fire_j(g):
        row0 = g * 16
        jv = idx_j[pl.ds(row0, 16)]
        for ln in range(16):
            pltpu.make_async_copy(
                jemb_hbm.at[jv[ln]], rows_j.at[row0 + ln], sem).start()

    @plsc.parallel_loop(0, BPW // 16)
    def fire_g(g):
        row0 = g * 16
        gv = idx_g[pl.ds(row0, 16)]
        for ln in range(16):
            pltpu.make_async_copy(
                gemb_hbm.at[gv[ln]], rows_g.at[row0 + ln], sem).start()

    pltpu.make_async_copy(jemb_hbm.at[pl.ds(0, BPW)], rows_j, sem).wait()
    pltpu.make_async_copy(gemb_hbm.at[pl.ds(0, BPW)], rows_g, sem).wait()

    def block(blk, carry):
        row0 = blk * BLK
        vx = jnp.zeros((16,), jnp.float32)
        vjj = jnp.zeros((16,), jnp.float32)
        vgg = jnp.zeros((16,), jnp.float32)
        for r in range(BLK):
            row = row0 + r
            px = jnp.zeros((16,), jnp.float32)
            pjj = jnp.zeros((16,), jnp.float32)
            pgg = jnp.zeros((16,), jnp.float32)
            for c in range(D // 16):
                jvv = rows_j[row, pl.ds(c * 16, 16)]
                gvv = rows_g[row, pl.ds(c * 16, 16)]
                px = px + jvv * gvv * wv[c]
                pjj = pjj + jvv * jvv
                pgg = pgg + gvv * gvv
            m = lane == r
            vx = jnp.where(m, lanesum(px), vx)
            vjj = jnp.where(m, lanesum(pjj), vjj)
            vgg = jnp.where(m, lanesum(pgg), vgg)
        out_v[pl.ds(row0, BLK)] = vx * _nrsqrt(vjj * vgg)
        return carry

    lax.fori_loop(0, BPW // BLK, block, 0)
    pltpu.sync_copy(out_v, out_hbm.at[pl.ds(base, BPW)])


def _tc_body(jidx_sm, gidx_sm, jemb_any, gemb_any, w_v, out_b,
             rows_j, rows_g, sem):
    i = pl.program_id(0)
    base = i * TCB
    cps = []
    for r in range(TCB):
        cps.append(pltpu.make_async_copy(
            jemb_any.at[jidx_sm[base + r]], rows_j.at[r], sem))
        cps.append(pltpu.make_async_copy(
            gemb_any.at[gidx_sm[base + r]], rows_g.at[r], sem))
    for cp in cps:
        cp.start()
    for cp in cps:
        cp.wait()
    j = rows_j[...]
    g = rows_g[...]
    x = jnp.sum(j * g * w_v[...], axis=1)
    n2 = jnp.sum(j * j, axis=1) * jnp.sum(g * g, axis=1)
    out_b[...] = (x * lax.rsqrt(n2))[:, None]


@functools.partial(jax.jit, static_argnums=())
def _gmf(job_f, geek_f, job_emb, geek_emb, w_f):
    mesh = plsc.VectorSubcoreMesh(core_axis_name="c", subcore_axis_name="s")
    sc_run = functools.partial(
        pl.kernel,
        out_type=jax.ShapeDtypeStruct((SPLIT,), jnp.float32),
        mesh=mesh,
        scratch_types=[
            pltpu.VMEM((BPW,), jnp.int32),
            pltpu.VMEM((BPW,), jnp.int32),
            pltpu.VMEM((BPW, D), jnp.float32),
            pltpu.VMEM((BPW, D), jnp.float32),
            pltpu.VMEM((D,), jnp.float32),
            pltpu.VMEM((BPW,), jnp.float32),
            pltpu.SemaphoreType.DMA,
        ],
    )(_sc_body)
    ntc = B - SPLIT
    out_tc = pl.pallas_call(
        _tc_body,
        grid=(ntc // TCB,),
        in_specs=[
            pl.BlockSpec(memory_space=pltpu.SMEM),
            pl.BlockSpec(memory_space=pltpu.SMEM),
            pl.BlockSpec(memory_space=pltpu.HBM),
            pl.BlockSpec(memory_space=pltpu.HBM),
            pl.BlockSpec((1, D), lambda i: (0, 0)),
        ],
        out_specs=pl.BlockSpec((TCB, 1), lambda i: (i, 0)),
        out_shape=jax.ShapeDtypeStruct((ntc, 1), jnp.float32),
        scratch_shapes=[
            pltpu.VMEM((TCB, D), jnp.float32),
            pltpu.VMEM((TCB, D), jnp.float32),
            pltpu.SemaphoreType.DMA,
        ],
    )(job_f[SPLIT:], geek_f[SPLIT:], job_emb, geek_emb,
      w_f.reshape(1, D))
    out_sc = sc_run(job_f[:SPLIT], geek_f[:SPLIT], job_emb, geek_emb, w_f)
    return out_sc, out_tc


def kernel(job, geek, job_emb, geek_emb, W):
    job_f = job.reshape(-1).astype(jnp.int32)
    geek_f = geek.reshape(-1).astype(jnp.int32)
    w_f = W.reshape(-1)
    out_sc, out_tc = _gmf(job_f, geek_f, job_emb, geek_emb, w_f)
    return jnp.concatenate([out_sc.reshape(SPLIT, 1), out_tc], axis=0)


# speedup vs baseline: 1.5745x; 1.1599x over previous
import functools

import jax
import jax.numpy as jnp
from jax import lax
from jax.experimental import pallas as pl
from jax.experimental.pallas import tpu as pltpu
from jax.experimental.pallas import tpu_sc as plsc

B = 16384
D = 64
NC = 2
NS = 16
NW = NC * NS
BPW = B // NW
NPASS = 2
PB = BPW // NPASS
BLK = 16


def _nrsqrt(y):
    i = lax.bitcast_convert_type(y, jnp.int32)
    i = jnp.int32(0x5F3759DF) - lax.shift_right_arithmetic(i, 1)
    x = lax.bitcast_convert_type(i, jnp.float32)
    for _ in range(3):
        x = x * (1.5 - 0.5 * y * x * x)
    return x


def _gmf_body(job_hbm, geek_hbm, jemb_hbm, gemb_hbm, w_hbm, out_hbm,
              idx_j, idx_g, rows_j, rows_g, w_v, out_v, sem):
    wid = lax.axis_index("s") * NC + lax.axis_index("c")
    base = wid * BPW

    pltpu.sync_copy(w_hbm, w_v)
    pltpu.sync_copy(job_hbm.at[pl.ds(base, BPW)], idx_j)
    pltpu.sync_copy(geek_hbm.at[pl.ds(base, BPW)], idx_g)

    wv = [w_v[pl.ds(c * 16, 16)] for c in range(D // 16)]
    lane = lax.iota(jnp.int32, 16)
    perms = [jnp.bitwise_xor(lane, s) for s in (8, 4, 2, 1)]

    gdn = lax.GatherDimensionNumbers(
        offset_dims=(), collapsed_slice_dims=(0,), start_index_map=(0,))

    def lanesum(v):
        for p in perms:
            v = v + lax.gather(v, p[:, None], gdn, (1,),
                               mode=lax.GatherScatterMode.PROMISE_IN_BOUNDS)
        return v

    def one_pass(p, pcarry):
        pbase = p * PB

        @plsc.parallel_loop(0, PB // 16)
        def fire_j(g):
            row0 = g * 16
            jv = idx_j[pl.ds(pbase + row0, 16)]
            for ln in range(16):
                pltpu.make_async_copy(
                    jemb_hbm.at[jv[ln]], rows_j.at[row0 + ln], sem).start()

        @plsc.parallel_loop(0, PB // 16)
        def fire_g(g):
            row0 = g * 16
            gv = idx_g[pl.ds(pbase + row0, 16)]
            for ln in range(16):
                pltpu.make_async_copy(
                    gemb_hbm.at[gv[ln]], rows_g.at[row0 + ln], sem).start()

        pltpu.make_async_copy(jemb_hbm.at[pl.ds(0, PB)], rows_j, sem).wait()
        pltpu.make_async_copy(gemb_hbm.at[pl.ds(0, PB)], rows_g, sem).wait()

        def block(blk, carry):
            row0 = blk * BLK
            vx = jnp.zeros((16,), jnp.float32)
            vjj = jnp.zeros((16,), jnp.float32)
            vgg = jnp.zeros((16,), jnp.float32)
            for r in range(BLK):
                row = row0 + r
                px = jnp.zeros((16,), jnp.float32)
                pjj = jnp.zeros((16,), jnp.float32)
                pgg = jnp.zeros((16,), jnp.float32)
                for c in range(D // 16):
                    jv = rows_j[row, pl.ds(c * 16, 16)]
                    gv = rows_g[row, pl.ds(c * 16, 16)]
                    px = px + jv * gv * wv[c]
                    pjj = pjj + jv * jv
                    pgg = pgg + gv * gv
                m = lane == r
                vx = jnp.where(m, lanesum(px), vx)
                vjj = jnp.where(m, lanesum(pjj), vjj)
                vgg = jnp.where(m, lanesum(pgg), vgg)
            out_v[pl.ds(pbase + row0, BLK)] = vx * _nrsqrt(vjj * vgg)
            return carry

        lax.fori_loop(0, PB // BLK, block, 0)
        return pcarry

    lax.fori_loop(0, NPASS, one_pass, 0)
    pltpu.sync_copy(out_v, out_hbm.at[pl.ds(base, BPW)])


@functools.partial(jax.jit, static_argnums=())
def _gmf(job_f, geek_f, job_emb, geek_emb, w_f):
    mesh = plsc.VectorSubcoreMesh(core_axis_name="c", subcore_axis_name="s")
    run = functools.partial(
        pl.kernel,
        out_type=jax.ShapeDtypeStruct((B,), jnp.float32),
        mesh=mesh,
        scratch_types=[
            pltpu.VMEM((BPW,), jnp.int32),
            pltpu.VMEM((BPW,), jnp.int32),
            pltpu.VMEM((PB, D), jnp.float32),
            pltpu.VMEM((PB, D), jnp.float32),
            pltpu.VMEM((D,), jnp.float32),
            pltpu.VMEM((BPW,), jnp.float32),
            pltpu.SemaphoreType.DMA,
        ],
    )(_gmf_body)
    return run(job_f, geek_f, job_emb, geek_emb, w_f)


def kernel(job, geek, job_emb, geek_emb, W):
    job_f = job.reshape(-1).astype(jnp.int32)
    geek_f = geek.reshape(-1).astype(jnp.int32)
    w_f = W.reshape(-1)
    out = _gmf(job_f, geek_f, job_emb, geek_emb, w_f)
    return out.reshape(B, 1)
